# baseline (device time: 124657 ns/iter reference)
import jax
import jax.numpy as jnp
from jax import lax
from jax.experimental import pallas as pl
from jax.experimental.pallas import tpu as pltpu


def kernel(x):
    m, n = x.shape
    half = n // 2
    out_dtype = jnp.bfloat16

    def body(x_ref, out_ref, stage_ref, send_sem, recv_sem):
        my_x = lax.axis_index("x")
        my_y = lax.axis_index("y")
        my_z = lax.axis_index("z")
        peer_y = 1 - my_y
        peer = (my_x, peer_y, my_z)

        barrier = pltpu.get_barrier_semaphore()
        pl.semaphore_signal(
            barrier, inc=1, device_id=peer,
            device_id_type=pl.DeviceIdType.MESH,
        )
        pl.semaphore_wait(barrier, 1)

        stage_ref[...] = x_ref[:, pl.ds(peer_y * half, half)].astype(out_dtype)

        h2 = m // 2
        rdmas = []
        for s in range(2):
            rdma = pltpu.make_async_remote_copy(
                src_ref=stage_ref.at[pl.ds(s * h2, h2), :],
                dst_ref=out_ref.at[pl.ds(my_y * m + s * h2, h2), :],
                send_sem=send_sem.at[s],
                recv_sem=recv_sem.at[s],
                device_id=peer,
                device_id_type=pl.DeviceIdType.MESH,
            )
            rdma.start()
            rdmas.append(rdma)

        out_ref[pl.ds(my_y * m, m), :] = (
            x_ref[:, pl.ds(my_y * half, half)].astype(out_dtype)
        )

        for rdma in rdmas:
            rdma.wait()

    return pl.pallas_call(
        body,
        out_shape=jax.ShapeDtypeStruct((2 * m, half), out_dtype),
        in_specs=[pl.BlockSpec(memory_space=pltpu.VMEM)],
        out_specs=pl.BlockSpec(memory_space=pltpu.VMEM),
        scratch_shapes=[
            pltpu.VMEM((m, half), out_dtype),
            pltpu.SemaphoreType.DMA((2,)),
            pltpu.SemaphoreType.DMA((2,)),
        ],
        compiler_params=pltpu.CompilerParams(
            collective_id=0,
            vmem_limit_bytes=100 * 1024 * 1024,
        ),
    )(x)


# device time: 111543 ns/iter; 1.1176x vs baseline; 1.1176x over previous
import jax
import jax.numpy as jnp
from jax import lax
from jax.experimental import pallas as pl
from jax.experimental.pallas import tpu as pltpu

NC = 4


def kernel(x):
    m, n = x.shape
    half = n // 2
    mc = m // NC
    out_dtype = jnp.bfloat16

    def body(x_hbm, out_hbm, xbuf, stage, lbuf, in_sems, lsems,
             send_sems, recv_sems):
        my_x = lax.axis_index("x")
        my_y = lax.axis_index("y")
        my_z = lax.axis_index("z")
        peer_y = 1 - my_y
        peer = (my_x, peer_y, my_z)

        barrier = pltpu.get_barrier_semaphore()
        pl.semaphore_signal(
            barrier, inc=1, device_id=peer,
            device_id_type=pl.DeviceIdType.MESH,
        )
        pl.semaphore_wait(barrier, 1)

        def in_dma(i, slot):
            return pltpu.make_async_copy(
                x_hbm.at[pl.ds(i * mc, mc), :],
                xbuf.at[slot],
                in_sems.at[slot],
            )

        in_dma(0, 0).start()
        in_dma(1, 1).start()

        rdmas = []
        lcopies = []
        for i in range(NC):
            slot = i % 2
            in_dma(i, slot).wait()

            stage[i] = xbuf[slot, :, pl.ds(peer_y * half, half)].astype(
                out_dtype
            )
            rdma = pltpu.make_async_remote_copy(
                src_ref=stage.at[i],
                dst_ref=out_hbm.at[pl.ds(my_y * m + i * mc, mc), :],
                send_sem=send_sems.at[i],
                recv_sem=recv_sems.at[i],
                device_id=peer,
                device_id_type=pl.DeviceIdType.MESH,
            )
            rdma.start()
            rdmas.append(rdma)

            if i >= 2:
                lcopies[i - 2].wait()
            lbuf[slot] = xbuf[slot, :, pl.ds(my_y * half, half)].astype(
                out_dtype
            )
            lcopy = pltpu.make_async_copy(
                lbuf.at[slot],
                out_hbm.at[pl.ds(my_y * m + i * mc, mc), :],
                lsems.at[slot],
            )
            lcopy.start()
            lcopies.append(lcopy)

            if i + 2 < NC:
                in_dma(i + 2, slot).start()

        for lcopy in lcopies[-2:]:
            lcopy.wait()
        for rdma in rdmas:
            rdma.wait()

    return pl.pallas_call(
        body,
        out_shape=jax.ShapeDtypeStruct((2 * m, half), out_dtype),
        in_specs=[pl.BlockSpec(memory_space=pl.ANY)],
        out_specs=pl.BlockSpec(memory_space=pl.ANY),
        scratch_shapes=[
            pltpu.VMEM((2, mc, n), x.dtype),
            pltpu.VMEM((NC, mc, half), out_dtype),
            pltpu.VMEM((2, mc, half), out_dtype),
            pltpu.SemaphoreType.DMA((2,)),
            pltpu.SemaphoreType.DMA((2,)),
            pltpu.SemaphoreType.DMA((NC,)),
            pltpu.SemaphoreType.DMA((NC,)),
        ],
        compiler_params=pltpu.CompilerParams(
            collective_id=0,
            vmem_limit_bytes=100 * 1024 * 1024,
        ),
    )(x)


# device time: 110044 ns/iter; 1.1328x vs baseline; 1.0136x over previous
import jax
import jax.numpy as jnp
from jax import lax
from jax.experimental import pallas as pl
from jax.experimental.pallas import tpu as pltpu

NC = 8


def kernel(x):
    m, n = x.shape
    half = n // 2
    mc = m // NC
    out_dtype = jnp.bfloat16

    def body(x_hbm, out_hbm, xbuf, stage, lbuf, in_sems, lsems,
             send_sems, recv_sems):
        my_x = lax.axis_index("x")
        my_y = lax.axis_index("y")
        my_z = lax.axis_index("z")
        peer_y = 1 - my_y
        peer = (my_x, peer_y, my_z)

        barrier = pltpu.get_barrier_semaphore()
        pl.semaphore_signal(
            barrier, inc=1, device_id=peer,
            device_id_type=pl.DeviceIdType.MESH,
        )
        pl.semaphore_wait(barrier, 1)

        def in_dma(i, slot):
            return pltpu.make_async_copy(
                x_hbm.at[pl.ds(i * mc, mc), :],
                xbuf.at[slot],
                in_sems.at[slot],
            )

        in_dma(0, 0).start()
        in_dma(1, 1).start()

        rdmas = []
        lcopies = []
        for i in range(NC):
            slot = i % 2
            in_dma(i, slot).wait()

            stage[i] = xbuf[slot, :, pl.ds(peer_y * half, half)].astype(
                out_dtype
            )
            rdma = pltpu.make_async_remote_copy(
                src_ref=stage.at[i],
                dst_ref=out_hbm.at[pl.ds(my_y * m + i * mc, mc), :],
                send_sem=send_sems.at[i],
                recv_sem=recv_sems.at[i],
                device_id=peer,
                device_id_type=pl.DeviceIdType.MESH,
            )
            rdma.start()
            rdmas.append(rdma)

            if i >= 2:
                lcopies[i - 2].wait()
            lbuf[slot] = xbuf[slot, :, pl.ds(my_y * half, half)].astype(
                out_dtype
            )
            lcopy = pltpu.make_async_copy(
                lbuf.at[slot],
                out_hbm.at[pl.ds(my_y * m + i * mc, mc), :],
                lsems.at[slot],
            )
            lcopy.start()
            lcopies.append(lcopy)

            if i + 2 < NC:
                in_dma(i + 2, slot).start()

        for lcopy in lcopies[-2:]:
            lcopy.wait()
        for rdma in rdmas:
            rdma.wait()

    return pl.pallas_call(
        body,
        out_shape=jax.ShapeDtypeStruct((2 * m, half), out_dtype),
        in_specs=[pl.BlockSpec(memory_space=pl.ANY)],
        out_specs=pl.BlockSpec(memory_space=pl.ANY),
        scratch_shapes=[
            pltpu.VMEM((2, mc, n), x.dtype),
            pltpu.VMEM((NC, mc, half), out_dtype),
            pltpu.VMEM((2, mc, half), out_dtype),
            pltpu.SemaphoreType.DMA((2,)),
            pltpu.SemaphoreType.DMA((2,)),
            pltpu.SemaphoreType.DMA((NC,)),
            pltpu.SemaphoreType.DMA((NC,)),
        ],
        compiler_params=pltpu.CompilerParams(
            collective_id=0,
            vmem_limit_bytes=100 * 1024 * 1024,
        ),
    )(x)
